# baseline (speedup 1.0000x reference)
import functools
import itertools as it

import jax
import jax.numpy as jnp
from jax import lax
from jax.experimental import pallas as pl
from jax.experimental.pallas import tpu as pltpu
from jax.experimental.pallas import tpu_sc as plsc

_K = 6
_SKIP = 4
_TRANSIENTS = 200
_P = 3

_B, _R, _T = 8, 16, 2048
_NROWS = _B * _R
_TOUT = _T - _TRANSIENTS
_NLIN = _K
_MONOMS = tuple(it.combinations_with_replacement(range(_NLIN), _P))
_NFEAT = 1 + _NLIN + len(_MONOMS)

_NWORKERS = 32
_ROWS_PER_W = _NROWS // _NWORKERS

_BLK = 16
_BLOCKS_PER_CHUNK = 29
_CHUNK_T = _BLOCKS_PER_CHUNK * _BLK
_NCHUNKS = 4
_XPAD = 2064
_STAGE = _CHUNK_T * _NFEAT
_PAGE = _TOUT * _NFEAT
_PAGE_PAD = 116480


def _chunk_words(c):
    n_t = min(_CHUNK_T, _TOUT - c * _CHUNK_T)
    return n_t * _NFEAT


def _body(x_hbm, out_hbm, xin, s0, s1, sem0, sem1):
    cid = lax.axis_index("c")
    sid = lax.axis_index("s")
    wid = sid * 2 + cid
    iota63 = lax.iota(jnp.int32, _BLK) * _NFEAT
    stages = (s0, s1)
    sems = (sem0, sem1)

    def row_body(rr, carry):
        r = wid * _ROWS_PER_W + rr
        pltpu.sync_copy(x_hbm.at[pl.ds(r * _T, _T)], xin.at[pl.ds(0, _T)])

        for c in range(_NCHUNKS):
            buf = c % 2
            stage, sem = stages[buf], sems[buf]
            nw = _chunk_words(c)
            prev_nw = _chunk_words(c - 2) if c >= 2 else _chunk_words(c + 2)

            def wait_prev(prev_nw=prev_nw, stage=stage, sem=sem):
                pltpu.make_async_copy(
                    stage.at[pl.ds(0, prev_nw)],
                    out_hbm.at[pl.ds(0, prev_nw)],
                    sem,
                ).wait()

            if c >= 2:
                wait_prev()
            else:
                @pl.when(rr > 0)
                def _():
                    wait_prev()

            def blk(tb, carry2, c=c, stage=stage):
                t0 = c * _CHUNK_T + tb * _BLK
                lin = [xin[pl.ds(t0 + 180 + _SKIP * j, _BLK)] for j in range(_NLIN)]
                pairs = {}
                for a in range(_NLIN):
                    for b in range(a, _NLIN):
                        pairs[(a, b)] = lin[a] * lin[b]
                ones = jnp.full((_BLK,), 1.0, dtype=jnp.float32)
                vbase = iota63 + tb * (_BLK * _NFEAT)
                plsc.store_scatter(stage, [vbase], ones)
                for j in range(_NLIN):
                    plsc.store_scatter(stage, [vbase + (1 + j)], lin[j])
                for m, (i, j, k) in enumerate(_MONOMS):
                    plsc.store_scatter(stage, [vbase + (1 + _NLIN + m)],
                                       pairs[(i, j)] * lin[k])
                return carry2

            lax.fori_loop(0, _BLOCKS_PER_CHUNK, blk, 0)
            pltpu.make_async_copy(
                stage.at[pl.ds(0, nw)],
                out_hbm.at[pl.ds(r * _PAGE_PAD + c * _STAGE, nw)],
                sem,
            ).start()
        return carry

    lax.fori_loop(0, _ROWS_PER_W, row_body, 0)
    pltpu.make_async_copy(
        s0.at[pl.ds(0, _chunk_words(2))],
        out_hbm.at[pl.ds(0, _chunk_words(2))], sem0).wait()
    pltpu.make_async_copy(
        s1.at[pl.ds(0, _chunk_words(3))],
        out_hbm.at[pl.ds(0, _chunk_words(3))], sem1).wait()


@functools.partial(jax.jit)
def kernel(X):
    Xf = X.reshape(_NROWS * _T)
    mesh = plsc.VectorSubcoreMesh(core_axis_name="c", subcore_axis_name="s")
    out = pl.kernel(
        _body,
        out_type=jax.ShapeDtypeStruct((_NROWS * _PAGE_PAD,), jnp.float32),
        mesh=mesh,
        compiler_params=pltpu.CompilerParams(needs_layout_passes=False),
        scratch_types=[
            pltpu.VMEM((_XPAD,), jnp.float32),
            pltpu.VMEM((_STAGE,), jnp.float32),
            pltpu.VMEM((_STAGE,), jnp.float32),
            pltpu.SemaphoreType.DMA,
            pltpu.SemaphoreType.DMA,
        ],
    )(Xf)
    out = out.reshape(_NROWS, _PAGE_PAD)[:, :_PAGE]
    return out.reshape(_B, _R, _TOUT, _NFEAT)


# speedup vs baseline: 1.3460x; 1.3460x over previous
import functools
import itertools as it

import jax
import jax.numpy as jnp
from jax import lax
from jax.experimental import pallas as pl
from jax.experimental.pallas import tpu as pltpu
from jax.experimental.pallas import tpu_sc as plsc

_K = 6
_SKIP = 4
_TRANSIENTS = 200
_P = 3

_B, _R, _T = 8, 16, 2048
_NROWS = _B * _R
_TOUT = _T - _TRANSIENTS
_NLIN = _K
_MONOMS = tuple(it.combinations_with_replacement(range(_NLIN), _P))
_NFEAT = 1 + _NLIN + len(_MONOMS)
_LANES = 128

_NWORKERS = 32
_ROWS_PER_W = _NROWS // _NWORKERS

_BLK = 16
_BLOCKS_PER_CHUNK = 29
_CHUNK_T = _BLOCKS_PER_CHUNK * _BLK
_NCHUNKS = 4
_XPAD = 2064
_STAGE = _CHUNK_T * _LANES
_PAGE = _TOUT * _LANES
_S63 = _BLK * _NFEAT + _BLK


def _chunk_words(c):
    n_t = min(_CHUNK_T, _TOUT - c * _CHUNK_T)
    return n_t * _LANES


def _body(x_hbm, out_hbm, xin, ba, bb, s0, s1, sem0, sem1):
    cid = lax.axis_index("c")
    sid = lax.axis_index("s")
    wid = sid * 2 + cid
    iota63 = lax.iota(jnp.int32, _BLK) * _NFEAT
    stages = (s0, s1)
    sems = (sem0, sem1)

    def scatter_block(tb, buf, c):
        t0 = c * _CHUNK_T + tb * _BLK
        lin = [xin[pl.ds(t0 + 180 + _SKIP * j, _BLK)] for j in range(_NLIN)]
        pairs = {}
        for a in range(_NLIN):
            for b in range(a, _NLIN):
                pairs[(a, b)] = lin[a] * lin[b]
        ones = jnp.full((_BLK,), 1.0, dtype=jnp.float32)
        plsc.store_scatter(buf, [iota63], ones)
        for j in range(_NLIN):
            plsc.store_scatter(buf, [iota63 + (1 + j)], lin[j])
        for m, (i, j, k) in enumerate(_MONOMS):
            plsc.store_scatter(buf, [iota63 + (1 + _NLIN + m)],
                               pairs[(i, j)] * lin[k])

    def repack_block(tb, buf, stage):
        base = tb * (_BLK * _LANES)
        for tt in range(_BLK):
            for g in range(4):
                v = buf[pl.ds(tt * _NFEAT + 16 * g, _BLK)]
                stage[pl.ds(base + tt * _LANES + 16 * g, _BLK)] = v

    def row_body(rr, carry):
        r = wid * _ROWS_PER_W + rr
        pltpu.sync_copy(x_hbm.at[pl.ds(r * _T, _T)], xin.at[pl.ds(0, _T)])

        for c in range(_NCHUNKS):
            dbuf = c % 2
            stage, sem = stages[dbuf], sems[dbuf]
            nw = _chunk_words(c)
            prev_nw = _chunk_words(c - 2) if c >= 2 else _chunk_words(c + 2)

            def wait_prev(prev_nw=prev_nw, stage=stage, sem=sem):
                pltpu.make_async_copy(
                    stage.at[pl.ds(0, prev_nw)],
                    out_hbm.at[pl.ds(0, prev_nw)],
                    sem,
                ).wait()

            if c >= 2:
                wait_prev()
            else:
                @pl.when(rr > 0)
                def _():
                    wait_prev()

            scatter_block(0, ba, c)

            def pair_body(i, carry2, c=c, stage=stage):
                scatter_block(2 * i + 1, bb, c)
                repack_block(2 * i, ba, stage)
                scatter_block(2 * i + 2, ba, c)
                repack_block(2 * i + 1, bb, stage)
                return carry2

            lax.fori_loop(0, (_BLOCKS_PER_CHUNK - 1) // 2, pair_body, 0)
            repack_block(_BLOCKS_PER_CHUNK - 1, ba, stage)

            pltpu.make_async_copy(
                stage.at[pl.ds(0, nw)],
                out_hbm.at[pl.ds(r * _PAGE + c * _STAGE, nw)],
                sem,
            ).start()
        return carry

    lax.fori_loop(0, _ROWS_PER_W, row_body, 0)
    pltpu.make_async_copy(
        s0.at[pl.ds(0, _chunk_words(2))],
        out_hbm.at[pl.ds(0, _chunk_words(2))], sem0).wait()
    pltpu.make_async_copy(
        s1.at[pl.ds(0, _chunk_words(3))],
        out_hbm.at[pl.ds(0, _chunk_words(3))], sem1).wait()


@functools.partial(jax.jit)
def kernel(X):
    Xf = X.reshape(_NROWS * _T)
    mesh = plsc.VectorSubcoreMesh(core_axis_name="c", subcore_axis_name="s")
    out = pl.kernel(
        _body,
        out_type=jax.ShapeDtypeStruct((_NROWS * _PAGE,), jnp.float32),
        mesh=mesh,
        compiler_params=pltpu.CompilerParams(needs_layout_passes=False),
        scratch_types=[
            pltpu.VMEM((_XPAD,), jnp.float32),
            pltpu.VMEM((_S63,), jnp.float32),
            pltpu.VMEM((_S63,), jnp.float32),
            pltpu.VMEM((_STAGE,), jnp.float32),
            pltpu.VMEM((_STAGE,), jnp.float32),
            pltpu.SemaphoreType.DMA,
            pltpu.SemaphoreType.DMA,
        ],
    )(Xf)
    out = out.reshape(_NROWS, _TOUT, _LANES)[:, :, :_NFEAT]
    return out.reshape(_B, _R, _TOUT, _NFEAT)
